# Initial kernel scaffold; baseline (speedup 1.0000x reference)
import jax
import jax.numpy as jnp
from jax.experimental import pallas as pl


def kernel(hidden_states, proto):
    raise NotImplementedError("write your pallas kernel here")


# speedup vs baseline: 1.8538x; 1.8538x over previous
import functools

import jax
import jax.numpy as jnp
from jax.experimental import pallas as pl
from jax.experimental.pallas import tpu as pltpu

NUM_EXPERTS = 64
TOP_K = 8
HIDDEN_SIZE = 2048
NUM_TOKENS = 16384

BT = 1024


def _router_body(h_ref, p_ref, w_ref, i_ref):
    h = h_ref[...]
    p = p_ref[...]
    hn = jnp.maximum(jnp.sqrt(jnp.sum(h * h, axis=1, keepdims=True)), 1e-12)
    pn = jnp.maximum(jnp.sqrt(jnp.sum(p * p, axis=1, keepdims=True)), 1e-12)
    hb = (h / hn).astype(jnp.bfloat16)
    pb = (p / pn).astype(jnp.bfloat16)
    logits = jax.lax.dot_general(
        hb, pb, (((1,), (1,)), ((), ())), preferred_element_type=jnp.float32
    )
    m = jnp.max(logits, axis=1, keepdims=True)
    e = jnp.exp(logits - m)
    probs = e / jnp.sum(e, axis=1, keepdims=True)

    iota = jax.lax.broadcasted_iota(jnp.int32, probs.shape, 1).astype(jnp.float32)
    col8 = jax.lax.broadcasted_iota(jnp.int32, (probs.shape[0], TOP_K), 1).astype(
        jnp.float32
    )
    x = probs
    acc_w = jnp.zeros((probs.shape[0], TOP_K), jnp.float32)
    acc_i = jnp.zeros((probs.shape[0], TOP_K), jnp.float32)
    for k in range(TOP_K):
        mk = jnp.max(x, axis=1, keepdims=True)
        imf = jnp.min(
            jnp.where(x == mk, iota, float(NUM_EXPERTS)), axis=1, keepdims=True
        )
        acc_w = acc_w + jnp.where(col8 == float(k), mk, 0.0)
        acc_i = acc_i + jnp.where(col8 == float(k), imf, 0.0)
        x = jnp.where(iota == imf, -1.0, x)
    w_ref[...] = acc_w
    i_ref[...] = acc_i.astype(jnp.int32)


@functools.partial(jax.jit, static_argnames=("interpret",))
def kernel(hidden_states, proto, interpret=False):
    grid = (NUM_TOKENS // BT,)
    return pl.pallas_call(
        _router_body,
        grid=grid,
        in_specs=[
            pl.BlockSpec((BT, HIDDEN_SIZE), lambda t: (t, 0)),
            pl.BlockSpec((NUM_EXPERTS, HIDDEN_SIZE), lambda t: (0, 0)),
        ],
        out_specs=[
            pl.BlockSpec((BT, TOP_K), lambda t: (t, 0)),
            pl.BlockSpec((BT, TOP_K), lambda t: (t, 0)),
        ],
        out_shape=[
            jax.ShapeDtypeStruct((NUM_TOKENS, TOP_K), jnp.float32),
            jax.ShapeDtypeStruct((NUM_TOKENS, TOP_K), jnp.int32),
        ],
        interpret=interpret,
    )(hidden_states, proto)


# speedup vs baseline: 1.8706x; 1.0090x over previous
import functools

import jax
import jax.numpy as jnp
from jax.experimental import pallas as pl
from jax.experimental.pallas import tpu as pltpu

NUM_EXPERTS = 64
TOP_K = 8
HIDDEN_SIZE = 2048
NUM_TOKENS = 16384

BT = 2048


def _proto_norm_body(p_ref, pb_ref):
    p = p_ref[...]
    pn = jnp.maximum(jnp.sqrt(jnp.sum(p * p, axis=1, keepdims=True)), 1e-12)
    pb_ref[...] = (p / pn).astype(jnp.bfloat16)


def _router_body(h_ref, pb_ref, w_ref, i_ref):
    h = h_ref[...]
    hn = jnp.maximum(jnp.sqrt(jnp.sum(h * h, axis=1, keepdims=True)), 1e-12)
    hb = (h / hn).astype(jnp.bfloat16)
    logits = jax.lax.dot_general(
        hb, pb_ref[...], (((1,), (1,)), ((), ())),
        preferred_element_type=jnp.float32,
    )
    m = jnp.max(logits, axis=1, keepdims=True)
    e = jnp.exp(logits - m)
    probs = e / jnp.sum(e, axis=1, keepdims=True)

    iota = jax.lax.broadcasted_iota(jnp.int32, probs.shape, 1).astype(jnp.float32)
    col8 = jax.lax.broadcasted_iota(jnp.int32, (probs.shape[0], TOP_K), 1).astype(
        jnp.float32
    )
    x = probs
    acc_w = jnp.zeros((probs.shape[0], TOP_K), jnp.float32)
    acc_i = jnp.zeros((probs.shape[0], TOP_K), jnp.float32)
    for k in range(TOP_K):
        mk = jnp.max(x, axis=1, keepdims=True)
        imf = jnp.min(
            jnp.where(x == mk, iota, float(NUM_EXPERTS)), axis=1, keepdims=True
        )
        acc_w = acc_w + jnp.where(col8 == float(k), mk, 0.0)
        acc_i = acc_i + jnp.where(col8 == float(k), imf, 0.0)
        x = jnp.where(iota == imf, -1.0, x)
    w_ref[...] = acc_w
    i_ref[...] = acc_i.astype(jnp.int32)


@jax.jit
def kernel(hidden_states, proto):
    proto_n = pl.pallas_call(
        _proto_norm_body,
        out_shape=jax.ShapeDtypeStruct((NUM_EXPERTS, HIDDEN_SIZE), jnp.bfloat16),
    )(proto)
    grid = (NUM_TOKENS // BT,)
    return pl.pallas_call(
        _router_body,
        grid=grid,
        in_specs=[
            pl.BlockSpec((BT, HIDDEN_SIZE), lambda t: (t, 0)),
            pl.BlockSpec((NUM_EXPERTS, HIDDEN_SIZE), lambda t: (0, 0)),
        ],
        out_specs=[
            pl.BlockSpec((BT, TOP_K), lambda t: (t, 0)),
            pl.BlockSpec((BT, TOP_K), lambda t: (t, 0)),
        ],
        out_shape=[
            jax.ShapeDtypeStruct((NUM_TOKENS, TOP_K), jnp.float32),
            jax.ShapeDtypeStruct((NUM_TOKENS, TOP_K), jnp.int32),
        ],
    )(hidden_states, proto_n)
